# baseline (speedup 1.0000x reference)
import jax
import jax.numpy as jnp
from jax.experimental import pallas as pl
from jax.experimental.pallas import tpu as pltpu

NDMA = 16
NCMP = 2
DEPTH = 16


def _linear_kernel(x_hbm, w_ref, b_ref, o_ref, xbuf, sems):
    B = x_hbm.shape[0]
    chd = B // NDMA
    chc = B // NCMP

    def start(i):
        pltpu.make_async_copy(
            x_hbm.at[pl.ds(i * chd, chd), :],
            xbuf.at[pl.ds(i * chd, chd), :],
            sems.at[i],
        ).start()

    for i in range(DEPTH):
        start(i)

    b_col = jnp.transpose(b_ref[...])
    per = NDMA // NCMP
    for c in range(NCMP):
        for s in range(c * per, (c + 1) * per):
            pltpu.make_async_copy(
                x_hbm.at[pl.ds(s * chd, chd), :],
                xbuf.at[pl.ds(s * chd, chd), :],
                sems.at[s],
            ).wait()
            if s + DEPTH < NDMA:
                start(s + DEPTH)
        acc = jax.lax.dot_general(
            xbuf[pl.ds(c * chc, chc), :], w_ref[...],
            dimension_numbers=(((1,), (1,)), ((), ())),
            preferred_element_type=jnp.float32,
        )
        o_ref[:, pl.ds(c * chc, chc)] = jnp.transpose(acc) + b_col


def kernel(x, W, b):
    B, K = x.shape
    N = W.shape[0]
    b2 = b.reshape(1, N)
    yt = pl.pallas_call(
        _linear_kernel,
        in_specs=[
            pl.BlockSpec(memory_space=pltpu.MemorySpace.HBM),
            pl.BlockSpec((N, K), lambda: (0, 0)),
            pl.BlockSpec((1, N), lambda: (0, 0)),
        ],
        out_specs=pl.BlockSpec((N, B), lambda: (0, 0)),
        out_shape=jax.ShapeDtypeStruct((N, B), x.dtype),
        scratch_shapes=[
            pltpu.VMEM((B, K), jnp.float32),
            pltpu.SemaphoreType.DMA((NDMA,)),
        ],
    )(x, W, b2)
    return yt.T


# speedup vs baseline: 1.0097x; 1.0097x over previous
import jax
import jax.numpy as jnp
from jax.experimental import pallas as pl
from jax.experimental.pallas import tpu as pltpu

NDMA = 32
NCMP = 4
DEPTH = 32


def _linear_kernel(x_hbm, w_ref, b_ref, o_ref, xbuf, sems):
    B = x_hbm.shape[0]
    chd = B // NDMA
    chc = B // NCMP

    def start(i):
        pltpu.make_async_copy(
            x_hbm.at[pl.ds(i * chd, chd), :],
            xbuf.at[pl.ds(i * chd, chd), :],
            sems.at[i],
        ).start()

    for i in range(DEPTH):
        start(i)

    b_col = jnp.transpose(b_ref[...])
    per = NDMA // NCMP
    for c in range(NCMP):
        for s in range(c * per, (c + 1) * per):
            pltpu.make_async_copy(
                x_hbm.at[pl.ds(s * chd, chd), :],
                xbuf.at[pl.ds(s * chd, chd), :],
                sems.at[s],
            ).wait()
            if s + DEPTH < NDMA:
                start(s + DEPTH)
        acc = jax.lax.dot_general(
            xbuf[pl.ds(c * chc, chc), :], w_ref[...],
            dimension_numbers=(((1,), (1,)), ((), ())),
            preferred_element_type=jnp.float32,
        )
        o_ref[:, pl.ds(c * chc, chc)] = jnp.transpose(acc) + b_col


def kernel(x, W, b):
    B, K = x.shape
    N = W.shape[0]
    b2 = b.reshape(1, N)
    yt = pl.pallas_call(
        _linear_kernel,
        in_specs=[
            pl.BlockSpec(memory_space=pltpu.MemorySpace.HBM),
            pl.BlockSpec((N, K), lambda: (0, 0)),
            pl.BlockSpec((1, N), lambda: (0, 0)),
        ],
        out_specs=pl.BlockSpec((N, B), lambda: (0, 0)),
        out_shape=jax.ShapeDtypeStruct((N, B), x.dtype),
        scratch_shapes=[
            pltpu.VMEM((B, K), jnp.float32),
            pltpu.SemaphoreType.DMA((NDMA,)),
        ],
    )(x, W, b2)
    return yt.T
